# baseline (speedup 1.0000x reference)
import functools

import jax
import jax.numpy as jnp
from jax import lax
from jax.experimental import pallas as pl
from jax.experimental.pallas import tpu as pltpu
from jax.experimental.pallas import tpu_sc as plsc

_N = 10000
_E = 320000
_D = 128
_G = 16
_DOUT = 64
_NC = 2
_NS = 16
_NW = _NC * _NS
_K = 128
_NCHUNK = 80
_EP = _NW * _NCHUNK * _K
_NACC = 10112
_ZROWS = _NACC // _NS
_WROWS = 624
_TAIL = _N - _NS * _WROWS
_BLK = 2000
_NBUF = 2
_NIB = 4
_NCHUNKP = _NCHUNK + 4

_sc_mesh = plsc.VectorSubcoreMesh(core_axis_name="c", subcore_axis_name="s")


@functools.partial(
    pl.kernel,
    out_type=jax.ShapeDtypeStruct((_NC, _N, _D), jnp.float32),
    mesh=_sc_mesh,
    scratch_types=[
        pltpu.VMEM((_NCHUNK, _K), jnp.int32),
        pltpu.VMEM((_K, _D), jnp.float32),
        pltpu.VMEM_SHARED((_NACC, _D), jnp.float32),
        pltpu.SemaphoreType.DMA,
    ],
)
def _deg_kernel(dstp_hbm, ones_hbm, zdeg_hbm, out_hbm, didx, ones_v, acc, sem):
    c = lax.axis_index("c")
    s = lax.axis_index("s")
    w = c * _NS + s
    pltpu.sync_copy(zdeg_hbm, acc.at[pl.ds(s * _ZROWS, _ZROWS)])
    pltpu.sync_copy(ones_hbm, ones_v)
    pltpu.sync_copy(dstp_hbm.at[w], didx)
    plsc.subcore_barrier()

    def fire(j, carry):
        pltpu.async_copy(ones_v, acc.at[didx.at[j]], sem, add=True)
        return carry

    lax.fori_loop(0, _NCHUNK, fire, 0)

    def drain(j, carry):
        pltpu.make_async_copy(ones_v, acc.at[didx.at[j]], sem).wait()
        return carry

    lax.fori_loop(0, _NCHUNK, drain, 0)
    plsc.subcore_barrier()
    pltpu.sync_copy(acc.at[pl.ds(s * _WROWS, _WROWS)],
                    out_hbm.at[c, pl.ds(s * _WROWS, _WROWS)])

    @pl.when(s == _NS - 1)
    def _():
        pltpu.sync_copy(acc.at[pl.ds(_NS * _WROWS, _TAIL)],
                        out_hbm.at[c, pl.ds(_NS * _WROWS, _TAIL)])


@functools.partial(
    pl.kernel,
    out_type=jax.ShapeDtypeStruct((_NC, _N, _D), jnp.float32),
    mesh=_sc_mesh,
    scratch_types=[
        pltpu.VMEM((_NIB, _K), jnp.int32),
        pltpu.VMEM((_NCHUNK, _K), jnp.int32),
        pltpu.VMEM((_NBUF, _K, _D), jnp.float32),
        pltpu.VMEM_SHARED((_NACC, _D), jnp.float32),
        pltpu.SemaphoreType.DMA,
        pltpu.SemaphoreType.DMA,
        pltpu.SemaphoreType.DMA,
        pltpu.SemaphoreType.DMA,
        pltpu.SemaphoreType.DMA,
        pltpu.SemaphoreType.DMA,
    ],
)
def _agg_kernel(g_hbm, srcp_hbm, dstp_hbm, zrow_hbm, out_hbm,
                sidx, didx, buf, acc, g0, g1, i0, i1, i2, i3):
    gsems = (g0, g1)
    isems = (i0, i1, i2, i3)
    c = lax.axis_index("c")
    s = lax.axis_index("s")
    w = c * _NS + s

    def iload(jj, sl):
        return pltpu.make_async_copy(srcp_hbm.at[w, jj], sidx.at[sl], isems[sl])

    def gather(jj, sl, b):
        return pltpu.make_async_copy(g_hbm.at[sidx.at[sl]], buf.at[b], gsems[b])

    pltpu.sync_copy(zrow_hbm, acc.at[pl.ds(s * _ZROWS, _ZROWS)])
    pltpu.sync_copy(dstp_hbm.at[w], didx)
    for m in range(3):
        iload(m, m).start()
    for b in range(_NBUF):
        iload(b, b).wait()
        gather(b, b, b).start()
    plsc.subcore_barrier()

    def outer(j0, carry):
        for b in range(_NIB):
            jj = j0 * _NIB + b
            gather(jj, b, b % _NBUF).wait()
            pltpu.sync_copy(buf.at[b % _NBUF], acc.at[didx.at[jj]], add=True)
            iload(jj + 2, (b + 2) % _NIB).wait()
            gather(jj + 2, (b + 2) % _NIB, b % _NBUF).start()
            iload(jj + 3, (b + 3) % _NIB).start()
        return carry

    lax.fori_loop(0, _NCHUNK // _NIB, outer, 0)
    gather(_NCHUNK, 0, 0).wait()
    gather(_NCHUNK + 1, 1, 1).wait()
    iload(_NCHUNK + 2, 2).wait()
    plsc.subcore_barrier()
    pltpu.sync_copy(acc.at[pl.ds(s * _WROWS, _WROWS)],
                    out_hbm.at[c, pl.ds(s * _WROWS, _WROWS)])

    @pl.when(s == _NS - 1)
    def _():
        pltpu.sync_copy(acc.at[pl.ds(_NS * _WROWS, _TAIL)],
                        out_hbm.at[c, pl.ds(_NS * _WROWS, _TAIL)])


def _dinv_from(degp):
    deg = degp[0, :, 0:1] + degp[1, :, 0:1] + 1.0
    return lax.rsqrt(deg)


def _tc_scale_body(x_ref, w_ref, degp_ref, o_ref):
    dinv = _dinv_from(degp_ref[...])
    h = jnp.dot(x_ref[...], w_ref[...], preferred_element_type=jnp.float32)
    o_ref[...] = h * dinv


def _tc_mid_body(sp_ref, g_ref, degp_ref, b_ref, w_ref, o_ref):
    dinv = _dinv_from(degp_ref[...])
    ssum = sp_ref[0] + sp_ref[1] + g_ref[...]
    h = jnp.maximum(ssum * dinv + b_ref[...], 0.0)
    o_ref[...] = jnp.dot(h, w_ref[...], preferred_element_type=jnp.float32) * dinv


def _tc_pool_body(sp_ref, g_ref, degp_ref, b_ref, batch_ref, wfc_ref, bfc_ref,
                  o_ref, sums_ref, cnt_ref):
    i = pl.program_id(0)
    dinv = _dinv_from(degp_ref[...])
    h = jnp.maximum((sp_ref[0] + sp_ref[1] + g_ref[...]) * dinv + b_ref[...], 0.0)
    gid = lax.broadcasted_iota(jnp.int32, (_BLK, _G), 1)
    onehot = (batch_ref[...] == gid).astype(jnp.float32)
    dn = (((0,), (0,)), ((), ()))
    ps = lax.dot_general(onehot, h, dn, preferred_element_type=jnp.float32)
    pc = lax.dot_general(onehot, jnp.ones((_BLK, _D), jnp.float32), dn,
                         preferred_element_type=jnp.float32)

    @pl.when(i == 0)
    def _():
        sums_ref[...] = jnp.zeros_like(sums_ref)
        cnt_ref[...] = jnp.zeros_like(cnt_ref)

    sums_ref[...] += ps
    cnt_ref[...] += pc
    pooled = sums_ref[...] / jnp.maximum(cnt_ref[...], 1.0)
    o_ref[...] = jnp.dot(pooled, wfc_ref[...],
                         preferred_element_type=jnp.float32) + bfc_ref[...]


def _tc_scale(x, W, degp):
    return pl.pallas_call(
        _tc_scale_body,
        grid=(_N // _BLK,),
        in_specs=[
            pl.BlockSpec((_BLK, _D), lambda i: (i, 0)),
            pl.BlockSpec((_D, _D), lambda i: (0, 0)),
            pl.BlockSpec((2, _BLK, _D), lambda i: (0, i, 0)),
        ],
        out_specs=pl.BlockSpec((_BLK, _D), lambda i: (i, 0)),
        out_shape=jax.ShapeDtypeStruct((_N, _D), jnp.float32),
    )(x, W, degp)


def _tc_mid(sp, g, degp, b, W):
    return pl.pallas_call(
        _tc_mid_body,
        grid=(_N // _BLK,),
        in_specs=[
            pl.BlockSpec((2, _BLK, _D), lambda i: (0, i, 0)),
            pl.BlockSpec((_BLK, _D), lambda i: (i, 0)),
            pl.BlockSpec((2, _BLK, _D), lambda i: (0, i, 0)),
            pl.BlockSpec((1, _D), lambda i: (0, 0)),
            pl.BlockSpec((_D, _D), lambda i: (0, 0)),
        ],
        out_specs=pl.BlockSpec((_BLK, _D), lambda i: (i, 0)),
        out_shape=jax.ShapeDtypeStruct((_N, _D), jnp.float32),
    )(sp, g, degp, b, W)


def _tc_pool(sp, g, degp, b, batchf, Wfc, bfc):
    return pl.pallas_call(
        _tc_pool_body,
        grid=(_N // _BLK,),
        in_specs=[
            pl.BlockSpec((2, _BLK, _D), lambda i: (0, i, 0)),
            pl.BlockSpec((_BLK, _D), lambda i: (i, 0)),
            pl.BlockSpec((2, _BLK, _D), lambda i: (0, i, 0)),
            pl.BlockSpec((1, _D), lambda i: (0, 0)),
            pl.BlockSpec((_BLK, 1), lambda i: (i, 0)),
            pl.BlockSpec((_D, _DOUT), lambda i: (0, 0)),
            pl.BlockSpec((1, _DOUT), lambda i: (0, 0)),
        ],
        out_specs=pl.BlockSpec((_G, _DOUT), lambda i: (0, 0)),
        out_shape=jax.ShapeDtypeStruct((_G, _DOUT), jnp.float32),
        scratch_shapes=[
            pltpu.VMEM((_G, _D), jnp.float32),
            pltpu.VMEM((_G, _D), jnp.float32),
        ],
    )(sp, g, degp, b, batchf, Wfc, bfc)


def kernel(x, edge_index, batch, W1, b1, W2, b2, Wfc, bfc):
    src = edge_index[0].astype(jnp.int32)
    dst = edge_index[1].astype(jnp.int32)
    pad = _EP - _E
    srcp = jnp.concatenate([src, jnp.zeros((pad,), jnp.int32)]).reshape(_NW, _NCHUNK, _K)
    dstp = jnp.concatenate([dst, jnp.full((pad,), _N, jnp.int32)]).reshape(_NW, _NCHUNK, _K)
    srcp = jnp.concatenate([srcp, jnp.zeros((_NW, _NBUF, _K), jnp.int32)], axis=1)

    ones = jnp.ones((_K, _D), jnp.float32)
    zrow = jnp.zeros((_ZROWS, _D), jnp.float32)

    degp = _deg_kernel(dstp, ones, zrow)

    g1 = _tc_scale(x, W1, degp)
    s1 = _agg_kernel(g1, srcp, dstp, zrow)
    g2 = _tc_mid(s1, g1, degp, b1.reshape(1, _D), W2)
    s2 = _agg_kernel(g2, srcp, dstp, zrow)

    batchf = batch.astype(jnp.int32).reshape(_N, 1)
    return _tc_pool(s2, g2, degp, b2.reshape(1, _D),
                    batchf, Wfc, bfc.reshape(1, _DOUT))


# speedup vs baseline: 1.1130x; 1.1130x over previous
import functools

import jax
import jax.numpy as jnp
from jax import lax
from jax.experimental import pallas as pl
from jax.experimental.pallas import tpu as pltpu
from jax.experimental.pallas import tpu_sc as plsc

_N = 10000
_E = 320000
_D = 128
_G = 16
_DOUT = 64
_NC = 2
_NS = 16
_NW = _NC * _NS
_K = 128
_NCHUNK = 80
_EP = _NW * _NCHUNK * _K
_NACC = 10112
_ZROWS = _NACC // _NS
_WROWS = 624
_TAIL = _N - _NS * _WROWS
_BLK = 2000
_NBUF = 2
_KB = 8
_NCHUNKP = _NCHUNK + 2 * _KB

_sc_mesh = plsc.VectorSubcoreMesh(core_axis_name="c", subcore_axis_name="s")


@functools.partial(
    pl.kernel,
    out_type=jax.ShapeDtypeStruct((_NC, _N, _D), jnp.float32),
    mesh=_sc_mesh,
    scratch_types=[
        pltpu.VMEM((_NCHUNK, _K), jnp.int32),
        pltpu.VMEM((_K, _D), jnp.float32),
        pltpu.VMEM_SHARED((_NACC, _D), jnp.float32),
        pltpu.SemaphoreType.DMA,
    ],
)
def _deg_kernel(dstp_hbm, ones_hbm, zdeg_hbm, out_hbm, didx, ones_v, acc, sem):
    c = lax.axis_index("c")
    s = lax.axis_index("s")
    w = c * _NS + s
    pltpu.sync_copy(zdeg_hbm, acc.at[pl.ds(s * _ZROWS, _ZROWS)])
    pltpu.sync_copy(ones_hbm, ones_v)
    pltpu.sync_copy(dstp_hbm.at[w], didx)
    plsc.subcore_barrier()

    def fire(j, carry):
        pltpu.async_copy(ones_v, acc.at[didx.at[j]], sem, add=True)
        return carry

    lax.fori_loop(0, _NCHUNK, fire, 0)

    def drain(j, carry):
        pltpu.make_async_copy(ones_v, acc.at[didx.at[j]], sem).wait()
        return carry

    lax.fori_loop(0, _NCHUNK, drain, 0)
    plsc.subcore_barrier()
    pltpu.sync_copy(acc.at[pl.ds(s * _WROWS, _WROWS)],
                    out_hbm.at[c, pl.ds(s * _WROWS, _WROWS)])

    @pl.when(s == _NS - 1)
    def _():
        pltpu.sync_copy(acc.at[pl.ds(_NS * _WROWS, _TAIL)],
                        out_hbm.at[c, pl.ds(_NS * _WROWS, _TAIL)])


@functools.partial(
    pl.kernel,
    out_type=jax.ShapeDtypeStruct((_NC, _N, _D), jnp.float32),
    mesh=_sc_mesh,
    scratch_types=[
        pltpu.VMEM((2, _KB, _K), jnp.int32),
        pltpu.VMEM((_NCHUNK, _K), jnp.int32),
        pltpu.VMEM((_NBUF, _K, _D), jnp.float32),
        pltpu.VMEM_SHARED((_NACC, _D), jnp.float32),
        pltpu.SemaphoreType.DMA,
        pltpu.SemaphoreType.DMA,
        pltpu.SemaphoreType.DMA,
        pltpu.SemaphoreType.DMA,
    ],
)
def _agg_kernel(g_hbm, srcp_hbm, dstp_hbm, zrow_hbm, out_hbm,
                sidxb, didx, buf, acc, g0, g1, i0, i1):
    gsems = (g0, g1)
    isems = (i0, i1)
    c = lax.axis_index("c")
    s = lax.axis_index("s")
    w = c * _NS + s

    def bank_load(bank, p):
        return pltpu.make_async_copy(
            srcp_hbm.at[w, pl.ds(bank * _KB, _KB)], sidxb.at[p], isems[p])

    def gather(p, slot, bb):
        return pltpu.make_async_copy(
            g_hbm.at[sidxb.at[p, slot]], buf.at[bb], gsems[bb])

    pltpu.sync_copy(zrow_hbm, acc.at[pl.ds(s * _ZROWS, _ZROWS)])
    pltpu.sync_copy(dstp_hbm.at[w], didx)
    pltpu.sync_copy(srcp_hbm.at[w, pl.ds(0, _KB)], sidxb.at[0])
    bank_load(1, 1).start()
    gather(0, 0, 0).start()
    plsc.subcore_barrier()

    def outer(j2, carry):
        for q in range(2):
            for b in range(_KB):
                jj = j2 * 2 * _KB + q * _KB + b
                bb = b % 2
                gather(q, b, bb).wait()
                if b < _KB - 1:
                    gather(q, b + 1, 1 - bb).start()
                else:
                    bank_load(j2 * 2 + q + 1, 1 - q).wait()
                    gather(1 - q, 0, 1 - bb).start()
                pltpu.sync_copy(buf.at[bb], acc.at[didx.at[jj]], add=True)
                if b == _KB - 1:
                    bank_load(j2 * 2 + q + 2, q).start()
        return carry

    lax.fori_loop(0, _NCHUNK // (2 * _KB), outer, 0)
    gather(0, 0, 0).wait()
    bank_load(11, 1).wait()
    plsc.subcore_barrier()
    pltpu.sync_copy(acc.at[pl.ds(s * _WROWS, _WROWS)],
                    out_hbm.at[c, pl.ds(s * _WROWS, _WROWS)])

    @pl.when(s == _NS - 1)
    def _():
        pltpu.sync_copy(acc.at[pl.ds(_NS * _WROWS, _TAIL)],
                        out_hbm.at[c, pl.ds(_NS * _WROWS, _TAIL)])


def _dinv_from(degp):
    deg = degp[0, :, 0:1] + degp[1, :, 0:1] + 1.0
    return lax.rsqrt(deg)


def _tc_scale_body(x_ref, w_ref, degp_ref, o_ref):
    dinv = _dinv_from(degp_ref[...])
    h = jnp.dot(x_ref[...], w_ref[...], preferred_element_type=jnp.float32)
    o_ref[...] = h * dinv


def _tc_mid_body(sp_ref, g_ref, degp_ref, b_ref, w_ref, o_ref):
    dinv = _dinv_from(degp_ref[...])
    ssum = sp_ref[0] + sp_ref[1] + g_ref[...]
    h = jnp.maximum(ssum * dinv + b_ref[...], 0.0)
    o_ref[...] = jnp.dot(h, w_ref[...], preferred_element_type=jnp.float32) * dinv


def _tc_pool_body(sp_ref, g_ref, degp_ref, b_ref, batch_ref, wfc_ref, bfc_ref,
                  o_ref, sums_ref, cnt_ref):
    i = pl.program_id(0)
    dinv = _dinv_from(degp_ref[...])
    h = jnp.maximum((sp_ref[0] + sp_ref[1] + g_ref[...]) * dinv + b_ref[...], 0.0)
    gid = lax.broadcasted_iota(jnp.int32, (_BLK, _G), 1)
    onehot = (batch_ref[...] == gid).astype(jnp.float32)
    dn = (((0,), (0,)), ((), ()))
    ps = lax.dot_general(onehot, h, dn, preferred_element_type=jnp.float32)
    pc = lax.dot_general(onehot, jnp.ones((_BLK, _D), jnp.float32), dn,
                         preferred_element_type=jnp.float32)

    @pl.when(i == 0)
    def _():
        sums_ref[...] = jnp.zeros_like(sums_ref)
        cnt_ref[...] = jnp.zeros_like(cnt_ref)

    sums_ref[...] += ps
    cnt_ref[...] += pc
    pooled = sums_ref[...] / jnp.maximum(cnt_ref[...], 1.0)
    o_ref[...] = jnp.dot(pooled, wfc_ref[...],
                         preferred_element_type=jnp.float32) + bfc_ref[...]


def _tc_scale(x, W, degp):
    return pl.pallas_call(
        _tc_scale_body,
        grid=(_N // _BLK,),
        in_specs=[
            pl.BlockSpec((_BLK, _D), lambda i: (i, 0)),
            pl.BlockSpec((_D, _D), lambda i: (0, 0)),
            pl.BlockSpec((2, _BLK, _D), lambda i: (0, i, 0)),
        ],
        out_specs=pl.BlockSpec((_BLK, _D), lambda i: (i, 0)),
        out_shape=jax.ShapeDtypeStruct((_N, _D), jnp.float32),
    )(x, W, degp)


def _tc_mid(sp, g, degp, b, W):
    return pl.pallas_call(
        _tc_mid_body,
        grid=(_N // _BLK,),
        in_specs=[
            pl.BlockSpec((2, _BLK, _D), lambda i: (0, i, 0)),
            pl.BlockSpec((_BLK, _D), lambda i: (i, 0)),
            pl.BlockSpec((2, _BLK, _D), lambda i: (0, i, 0)),
            pl.BlockSpec((1, _D), lambda i: (0, 0)),
            pl.BlockSpec((_D, _D), lambda i: (0, 0)),
        ],
        out_specs=pl.BlockSpec((_BLK, _D), lambda i: (i, 0)),
        out_shape=jax.ShapeDtypeStruct((_N, _D), jnp.float32),
    )(sp, g, degp, b, W)


def _tc_pool(sp, g, degp, b, batchf, Wfc, bfc):
    return pl.pallas_call(
        _tc_pool_body,
        grid=(_N // _BLK,),
        in_specs=[
            pl.BlockSpec((2, _BLK, _D), lambda i: (0, i, 0)),
            pl.BlockSpec((_BLK, _D), lambda i: (i, 0)),
            pl.BlockSpec((2, _BLK, _D), lambda i: (0, i, 0)),
            pl.BlockSpec((1, _D), lambda i: (0, 0)),
            pl.BlockSpec((_BLK, 1), lambda i: (i, 0)),
            pl.BlockSpec((_D, _DOUT), lambda i: (0, 0)),
            pl.BlockSpec((1, _DOUT), lambda i: (0, 0)),
        ],
        out_specs=pl.BlockSpec((_G, _DOUT), lambda i: (0, 0)),
        out_shape=jax.ShapeDtypeStruct((_G, _DOUT), jnp.float32),
        scratch_shapes=[
            pltpu.VMEM((_G, _D), jnp.float32),
            pltpu.VMEM((_G, _D), jnp.float32),
        ],
    )(sp, g, degp, b, batchf, Wfc, bfc)


def kernel(x, edge_index, batch, W1, b1, W2, b2, Wfc, bfc):
    src = edge_index[0].astype(jnp.int32)
    dst = edge_index[1].astype(jnp.int32)
    pad = _EP - _E
    srcp = jnp.concatenate([src, jnp.zeros((pad,), jnp.int32)]).reshape(_NW, _NCHUNK, _K)
    dstp = jnp.concatenate([dst, jnp.full((pad,), _N, jnp.int32)]).reshape(_NW, _NCHUNK, _K)
    srcp = jnp.concatenate(
        [srcp, jnp.zeros((_NW, _NCHUNKP - _NCHUNK, _K), jnp.int32)], axis=1)

    ones = jnp.ones((_K, _D), jnp.float32)
    zrow = jnp.zeros((_ZROWS, _D), jnp.float32)

    degp = _deg_kernel(dstp, ones, zrow)

    g1 = _tc_scale(x, W1, degp)
    s1 = _agg_kernel(g1, srcp, dstp, zrow)
    g2 = _tc_mid(s1, g1, degp, b1.reshape(1, _D), W2)
    s2 = _agg_kernel(g2, srcp, dstp, zrow)

    batchf = batch.astype(jnp.int32).reshape(_N, 1)
    return _tc_pool(s2, g2, degp, b2.reshape(1, _D),
                    batchf, Wfc, bfc.reshape(1, _DOUT))


# speedup vs baseline: 1.4465x; 1.2997x over previous
import functools

import jax
import jax.numpy as jnp
from jax import lax
from jax.experimental import pallas as pl
from jax.experimental.pallas import tpu as pltpu
from jax.experimental.pallas import tpu_sc as plsc

_N = 10000
_E = 320000
_D = 128
_G = 16
_DOUT = 64
_NC = 2
_NS = 16
_NW = _NC * _NS
_K = 128
_NCHUNK = 80
_EP = _NW * _NCHUNK * _K
_NACC = 10112
_ZROWS = _NACC // _NS
_WROWS = 624
_TAIL = _N - _NS * _WROWS
_BLK = 2000
_NBUF = 2
_KB = 8
_NCHUNKP = _NCHUNK + 2 * _KB

_sc_mesh = plsc.VectorSubcoreMesh(core_axis_name="c", subcore_axis_name="s")


@functools.partial(
    pl.kernel,
    out_type=jax.ShapeDtypeStruct((_NC, _N, _D), jnp.float32),
    mesh=_sc_mesh,
    scratch_types=[
        pltpu.VMEM((_NCHUNK, _K), jnp.int32),
        pltpu.VMEM((_K, _D), jnp.float32),
        pltpu.VMEM_SHARED((_NACC, _D), jnp.float32),
        pltpu.SemaphoreType.DMA,
    ],
)
def _deg_kernel(dstp_hbm, ones_hbm, zdeg_hbm, out_hbm, didx, ones_v, acc, sem):
    c = lax.axis_index("c")
    s = lax.axis_index("s")
    w = c * _NS + s
    pltpu.sync_copy(zdeg_hbm, acc.at[pl.ds(s * _ZROWS, _ZROWS)])
    pltpu.sync_copy(ones_hbm, ones_v)
    pltpu.sync_copy(dstp_hbm.at[w], didx)
    plsc.subcore_barrier()

    def fire(j, carry):
        pltpu.async_copy(ones_v, acc.at[didx.at[j]], sem, add=True)
        return carry

    lax.fori_loop(0, _NCHUNK, fire, 0)

    def drain(j, carry):
        pltpu.make_async_copy(ones_v, acc.at[didx.at[j]], sem).wait()
        return carry

    lax.fori_loop(0, _NCHUNK, drain, 0)
    plsc.subcore_barrier()
    pltpu.sync_copy(acc.at[pl.ds(s * _WROWS, _WROWS)],
                    out_hbm.at[c, pl.ds(s * _WROWS, _WROWS)])

    @pl.when(s == _NS - 1)
    def _():
        pltpu.sync_copy(acc.at[pl.ds(_NS * _WROWS, _TAIL)],
                        out_hbm.at[c, pl.ds(_NS * _WROWS, _TAIL)])


@functools.partial(
    pl.kernel,
    out_type=jax.ShapeDtypeStruct((_NC, _N, _D), jnp.float32),
    mesh=_sc_mesh,
    scratch_types=[
        pltpu.VMEM((_NCHUNK, _K), jnp.int32),
        pltpu.VMEM((_NCHUNK, _K), jnp.int32),
        pltpu.VMEM((_K, _D), jnp.float32),
        pltpu.VMEM_SHARED((_NACC, _D), jnp.float32),
    ],
)
def _agg_kernel(g_hbm, srcp_hbm, dstp_hbm, zrow_hbm, out_hbm,
                sidx, didx, buf, acc):
    c = lax.axis_index("c")
    s = lax.axis_index("s")
    w = c * _NS + s
    pltpu.sync_copy(zrow_hbm, acc.at[pl.ds(s * _ZROWS, _ZROWS)])
    pltpu.sync_copy(srcp_hbm.at[w], sidx)
    pltpu.sync_copy(dstp_hbm.at[w], didx)
    plsc.subcore_barrier()

    def step(j, carry):
        pltpu.sync_copy(g_hbm.at[sidx.at[j]], buf)
        pltpu.sync_copy(buf, acc.at[didx.at[j]], add=True)
        return carry

    lax.fori_loop(0, _NCHUNK, step, 0)
    plsc.subcore_barrier()
    pltpu.sync_copy(acc.at[pl.ds(s * _WROWS, _WROWS)],
                    out_hbm.at[c, pl.ds(s * _WROWS, _WROWS)])

    @pl.when(s == _NS - 1)
    def _():
        pltpu.sync_copy(acc.at[pl.ds(_NS * _WROWS, _TAIL)],
                        out_hbm.at[c, pl.ds(_NS * _WROWS, _TAIL)])


def _dinv_from(degp):
    deg = degp[0, :, 0:1] + degp[1, :, 0:1] + 1.0
    return lax.rsqrt(deg)


def _tc_scale_body(x_ref, w_ref, degp_ref, o_ref):
    dinv = _dinv_from(degp_ref[...])
    h = jnp.dot(x_ref[...], w_ref[...], preferred_element_type=jnp.float32)
    o_ref[...] = h * dinv


def _tc_mid_body(sp_ref, g_ref, degp_ref, b_ref, w_ref, o_ref):
    dinv = _dinv_from(degp_ref[...])
    ssum = sp_ref[0] + sp_ref[1] + g_ref[...]
    h = jnp.maximum(ssum * dinv + b_ref[...], 0.0)
    o_ref[...] = jnp.dot(h, w_ref[...], preferred_element_type=jnp.float32) * dinv


def _tc_pool_body(sp_ref, g_ref, degp_ref, b_ref, batch_ref, wfc_ref, bfc_ref,
                  o_ref, sums_ref, cnt_ref):
    i = pl.program_id(0)
    dinv = _dinv_from(degp_ref[...])
    h = jnp.maximum((sp_ref[0] + sp_ref[1] + g_ref[...]) * dinv + b_ref[...], 0.0)
    gid = lax.broadcasted_iota(jnp.int32, (_BLK, _G), 1)
    onehot = (batch_ref[...] == gid).astype(jnp.float32)
    dn = (((0,), (0,)), ((), ()))
    ps = lax.dot_general(onehot, h, dn, preferred_element_type=jnp.float32)
    pc = lax.dot_general(onehot, jnp.ones((_BLK, _D), jnp.float32), dn,
                         preferred_element_type=jnp.float32)

    @pl.when(i == 0)
    def _():
        sums_ref[...] = jnp.zeros_like(sums_ref)
        cnt_ref[...] = jnp.zeros_like(cnt_ref)

    sums_ref[...] += ps
    cnt_ref[...] += pc
    pooled = sums_ref[...] / jnp.maximum(cnt_ref[...], 1.0)
    o_ref[...] = jnp.dot(pooled, wfc_ref[...],
                         preferred_element_type=jnp.float32) + bfc_ref[...]


def _tc_scale(x, W, degp):
    return pl.pallas_call(
        _tc_scale_body,
        grid=(_N // _BLK,),
        in_specs=[
            pl.BlockSpec((_BLK, _D), lambda i: (i, 0)),
            pl.BlockSpec((_D, _D), lambda i: (0, 0)),
            pl.BlockSpec((2, _BLK, _D), lambda i: (0, i, 0)),
        ],
        out_specs=pl.BlockSpec((_BLK, _D), lambda i: (i, 0)),
        out_shape=jax.ShapeDtypeStruct((_N, _D), jnp.float32),
    )(x, W, degp)


def _tc_mid(sp, g, degp, b, W):
    return pl.pallas_call(
        _tc_mid_body,
        grid=(_N // _BLK,),
        in_specs=[
            pl.BlockSpec((2, _BLK, _D), lambda i: (0, i, 0)),
            pl.BlockSpec((_BLK, _D), lambda i: (i, 0)),
            pl.BlockSpec((2, _BLK, _D), lambda i: (0, i, 0)),
            pl.BlockSpec((1, _D), lambda i: (0, 0)),
            pl.BlockSpec((_D, _D), lambda i: (0, 0)),
        ],
        out_specs=pl.BlockSpec((_BLK, _D), lambda i: (i, 0)),
        out_shape=jax.ShapeDtypeStruct((_N, _D), jnp.float32),
    )(sp, g, degp, b, W)


def _tc_pool(sp, g, degp, b, batchf, Wfc, bfc):
    return pl.pallas_call(
        _tc_pool_body,
        grid=(_N // _BLK,),
        in_specs=[
            pl.BlockSpec((2, _BLK, _D), lambda i: (0, i, 0)),
            pl.BlockSpec((_BLK, _D), lambda i: (i, 0)),
            pl.BlockSpec((2, _BLK, _D), lambda i: (0, i, 0)),
            pl.BlockSpec((1, _D), lambda i: (0, 0)),
            pl.BlockSpec((_BLK, 1), lambda i: (i, 0)),
            pl.BlockSpec((_D, _DOUT), lambda i: (0, 0)),
            pl.BlockSpec((1, _DOUT), lambda i: (0, 0)),
        ],
        out_specs=pl.BlockSpec((_G, _DOUT), lambda i: (0, 0)),
        out_shape=jax.ShapeDtypeStruct((_G, _DOUT), jnp.float32),
        scratch_shapes=[
            pltpu.VMEM((_G, _D), jnp.float32),
            pltpu.VMEM((_G, _D), jnp.float32),
        ],
    )(sp, g, degp, b, batchf, Wfc, bfc)


def kernel(x, edge_index, batch, W1, b1, W2, b2, Wfc, bfc):
    src = edge_index[0].astype(jnp.int32)
    dst = edge_index[1].astype(jnp.int32)
    pad = _EP - _E
    srcp = jnp.concatenate([src, jnp.zeros((pad,), jnp.int32)]).reshape(_NW, _NCHUNK, _K)
    dstp = jnp.concatenate([dst, jnp.full((pad,), _N, jnp.int32)]).reshape(_NW, _NCHUNK, _K)

    ones = jnp.ones((_K, _D), jnp.float32)
    zrow = jnp.zeros((_ZROWS, _D), jnp.float32)

    degp = _deg_kernel(dstp, ones, zrow)

    g1 = _tc_scale(x, W1, degp)
    s1 = _agg_kernel(g1, srcp, dstp, zrow)
    g2 = _tc_mid(s1, g1, degp, b1.reshape(1, _D), W2)
    s2 = _agg_kernel(g2, srcp, dstp, zrow)

    batchf = batch.astype(jnp.int32).reshape(_N, 1)
    return _tc_pool(s2, g2, degp, b2.reshape(1, _D),
                    batchf, Wfc, bfc.reshape(1, _DOUT))


# speedup vs baseline: 1.4467x; 1.0001x over previous
import functools

import jax
import jax.numpy as jnp
from jax import lax
from jax.experimental import pallas as pl
from jax.experimental.pallas import tpu as pltpu
from jax.experimental.pallas import tpu_sc as plsc

_N = 10000
_E = 320000
_D = 128
_G = 16
_DOUT = 64
_NC = 2
_NS = 16
_NW = _NC * _NS
_K = 128
_NCHUNK = 80
_EP = _NW * _NCHUNK * _K
_NACC = 10112
_ZROWS = _NACC // _NS
_WROWS = 624
_TAIL = _N - _NS * _WROWS
_BLK = 2000
_NBUF = 2
_KB = 8
_NCHUNKP = _NCHUNK + 2 * _KB

_sc_mesh = plsc.VectorSubcoreMesh(core_axis_name="c", subcore_axis_name="s")


@functools.partial(
    pl.kernel,
    out_type=jax.ShapeDtypeStruct((_NC, _N, _D), jnp.float32),
    mesh=_sc_mesh,
    scratch_types=[
        pltpu.VMEM((_NCHUNK, _K), jnp.int32),
        pltpu.VMEM((_K, _D), jnp.float32),
        pltpu.VMEM_SHARED((_NACC, _D), jnp.float32),
        pltpu.SemaphoreType.DMA,
    ],
)
def _deg_kernel(dstp_hbm, ones_hbm, zdeg_hbm, out_hbm, didx, ones_v, acc, sem):
    c = lax.axis_index("c")
    s = lax.axis_index("s")
    w = c * _NS + s
    pltpu.sync_copy(zdeg_hbm, acc.at[pl.ds(s * _ZROWS, _ZROWS)])
    pltpu.sync_copy(ones_hbm, ones_v)
    pltpu.sync_copy(dstp_hbm.at[w], didx)
    plsc.subcore_barrier()

    def fire(j, carry):
        pltpu.async_copy(ones_v, acc.at[didx.at[j]], sem, add=True)
        return carry

    lax.fori_loop(0, _NCHUNK, fire, 0)

    def drain(j, carry):
        pltpu.make_async_copy(ones_v, acc.at[didx.at[j]], sem).wait()
        return carry

    lax.fori_loop(0, _NCHUNK, drain, 0)
    plsc.subcore_barrier()
    pltpu.sync_copy(acc.at[pl.ds(s * _WROWS, _WROWS)],
                    out_hbm.at[c, pl.ds(s * _WROWS, _WROWS)])

    @pl.when(s == _NS - 1)
    def _():
        pltpu.sync_copy(acc.at[pl.ds(_NS * _WROWS, _TAIL)],
                        out_hbm.at[c, pl.ds(_NS * _WROWS, _TAIL)])


@functools.partial(
    pl.kernel,
    out_type=jax.ShapeDtypeStruct((_NC, _N, _D), jnp.float32),
    mesh=_sc_mesh,
    scratch_types=[
        pltpu.VMEM((_NCHUNK, _K), jnp.int32),
        pltpu.VMEM((_NCHUNK, _K), jnp.int32),
        pltpu.VMEM((_K, _D), jnp.float32),
        pltpu.VMEM_SHARED((_NACC, _D), jnp.float32),
    ],
)
def _agg_kernel(g_hbm, srcp_hbm, dstp_hbm, zrow_hbm, out_hbm,
                sidx, didx, buf, acc):
    c = lax.axis_index("c")
    s = lax.axis_index("s")
    w = c * _NS + s
    pltpu.sync_copy(zrow_hbm, acc.at[pl.ds(s * _ZROWS, _ZROWS)])
    pltpu.sync_copy(srcp_hbm.at[w], sidx)
    pltpu.sync_copy(dstp_hbm.at[w], didx)
    plsc.subcore_barrier()

    def step(j, carry):
        pltpu.sync_copy(g_hbm.at[sidx.at[j]], buf)
        pltpu.sync_copy(buf, acc.at[didx.at[j]], add=True)
        return carry

    lax.fori_loop(0, _NCHUNK, step, 0)
    plsc.subcore_barrier()
    pltpu.sync_copy(acc.at[pl.ds(s * _WROWS, _WROWS)],
                    out_hbm.at[c, pl.ds(s * _WROWS, _WROWS)])

    @pl.when(s == _NS - 1)
    def _():
        pltpu.sync_copy(acc.at[pl.ds(_NS * _WROWS, _TAIL)],
                        out_hbm.at[c, pl.ds(_NS * _WROWS, _TAIL)])


def _dinv_from(degp):
    deg = degp[0, :, 0:1] + degp[1, :, 0:1] + 1.0
    return lax.rsqrt(deg)


def _tc_scale_body(x_ref, w_ref, degp_ref, o_ref):
    dinv = _dinv_from(degp_ref[...])
    h = jnp.dot(x_ref[...], w_ref[...], preferred_element_type=jnp.float32)
    o_ref[...] = h * dinv


def _tc_mid_body(sp_ref, g_ref, degp_ref, b_ref, w_ref, o_ref):
    dinv = _dinv_from(degp_ref[...])
    ssum = sp_ref[0] + sp_ref[1] + g_ref[...]
    h = jnp.maximum(ssum * dinv + b_ref[...], 0.0)
    o_ref[...] = jnp.dot(h, w_ref[...], preferred_element_type=jnp.float32) * dinv


def _tc_pool_body(sp_ref, g_ref, degp_ref, b_ref, batch_ref, wfc_ref, bfc_ref,
                  o_ref, sums_ref, cnt_ref):
    i = pl.program_id(0)
    dinv = _dinv_from(degp_ref[...])
    h = jnp.maximum((sp_ref[0] + sp_ref[1] + g_ref[...]) * dinv + b_ref[...], 0.0)
    gid = lax.broadcasted_iota(jnp.int32, (_BLK, _G), 1)
    onehot = (batch_ref[...] == gid).astype(jnp.float32)
    dn = (((0,), (0,)), ((), ()))
    ps = lax.dot_general(onehot, h, dn, preferred_element_type=jnp.float32)
    pc = lax.dot_general(onehot, jnp.ones((_BLK, _D), jnp.float32), dn,
                         preferred_element_type=jnp.float32)

    @pl.when(i == 0)
    def _():
        sums_ref[...] = jnp.zeros_like(sums_ref)
        cnt_ref[...] = jnp.zeros_like(cnt_ref)

    sums_ref[...] += ps
    cnt_ref[...] += pc
    pooled = sums_ref[...] / jnp.maximum(cnt_ref[...], 1.0)
    o_ref[...] = jnp.dot(pooled, wfc_ref[...],
                         preferred_element_type=jnp.float32) + bfc_ref[...]


def _tc_scale(x, W, degp):
    return pl.pallas_call(
        _tc_scale_body,
        grid=(_N // _BLK,),
        in_specs=[
            pl.BlockSpec((_BLK, _D), lambda i: (i, 0)),
            pl.BlockSpec((_D, _D), lambda i: (0, 0)),
            pl.BlockSpec((2, _BLK, _D), lambda i: (0, i, 0)),
        ],
        out_specs=pl.BlockSpec((_BLK, _D), lambda i: (i, 0)),
        out_shape=jax.ShapeDtypeStruct((_N, _D), jnp.float32),
    )(x, W, degp)


def _tc_mid(sp, g, degp, b, W):
    return pl.pallas_call(
        _tc_mid_body,
        grid=(_N // _BLK,),
        in_specs=[
            pl.BlockSpec((2, _BLK, _D), lambda i: (0, i, 0)),
            pl.BlockSpec((_BLK, _D), lambda i: (i, 0)),
            pl.BlockSpec((2, _BLK, _D), lambda i: (0, i, 0)),
            pl.BlockSpec((1, _D), lambda i: (0, 0)),
            pl.BlockSpec((_D, _D), lambda i: (0, 0)),
        ],
        out_specs=pl.BlockSpec((_BLK, _D), lambda i: (i, 0)),
        out_shape=jax.ShapeDtypeStruct((_N, _D), jnp.float32),
    )(sp, g, degp, b, W)


def _tc_pool(sp, g, degp, b, batchf, Wfc, bfc):
    return pl.pallas_call(
        _tc_pool_body,
        grid=(_N // _BLK,),
        in_specs=[
            pl.BlockSpec((2, _BLK, _D), lambda i: (0, i, 0)),
            pl.BlockSpec((_BLK, _D), lambda i: (i, 0)),
            pl.BlockSpec((2, _BLK, _D), lambda i: (0, i, 0)),
            pl.BlockSpec((1, _D), lambda i: (0, 0)),
            pl.BlockSpec((_BLK, 1), lambda i: (i, 0)),
            pl.BlockSpec((_D, _DOUT), lambda i: (0, 0)),
            pl.BlockSpec((1, _DOUT), lambda i: (0, 0)),
        ],
        out_specs=pl.BlockSpec((_G, _DOUT), lambda i: (0, 0)),
        out_shape=jax.ShapeDtypeStruct((_G, _DOUT), jnp.float32),
        scratch_shapes=[
            pltpu.VMEM((_G, _D), jnp.float32),
            pltpu.VMEM((_G, _D), jnp.float32),
        ],
    )(sp, g, degp, b, batchf, Wfc, bfc)


def kernel(x, edge_index, batch, W1, b1, W2, b2, Wfc, bfc):
    src = edge_index[0].astype(jnp.int32)
    dst = edge_index[1].astype(jnp.int32)
    pad = _EP - _E
    dst_pad = _N + jnp.arange(pad, dtype=jnp.int32) % (_NACC - _N)
    srcp = jnp.concatenate([src, jnp.zeros((pad,), jnp.int32)]).reshape(_NW, _NCHUNK, _K)
    dstp = jnp.concatenate([dst, dst_pad]).reshape(_NW, _NCHUNK, _K)

    ones = jnp.ones((_K, _D), jnp.float32)
    zrow = jnp.zeros((_ZROWS, _D), jnp.float32)

    degp = _deg_kernel(dstp, ones, zrow)

    g1 = _tc_scale(x, W1, degp)
    s1 = _agg_kernel(g1, srcp, dstp, zrow)
    g2 = _tc_mid(s1, g1, degp, b1.reshape(1, _D), W2)
    s2 = _agg_kernel(g2, srcp, dstp, zrow)

    batchf = batch.astype(jnp.int32).reshape(_N, 1)
    return _tc_pool(s2, g2, degp, b2.reshape(1, _D),
                    batchf, Wfc, bfc.reshape(1, _DOUT))
